# baseline (speedup 1.0000x reference)
import functools

import jax
import jax.numpy as jnp
from jax import lax
from jax.experimental import pallas as pl
from jax.experimental.pallas import tpu as pltpu
from jax.experimental.pallas import tpu_sc as plsc

N = 10000
NP = 10240
G = 64
E = 320000
EP = 327680
CH = 128
NCH_MAIN = 80
NCH_DEG = 160
K = 4
NGRP = NCH_MAIN // K
RPT = NP // 16

_mesh = plsc.VectorSubcoreMesh(core_axis_name="c", subcore_axis_name="s")
_sc_params = pltpu.CompilerParams(use_tc_tiling_on_sc=False)


@functools.partial(
    pl.kernel,
    out_type=jax.ShapeDtypeStruct((2, NP), jnp.float32),
    mesh=_mesh,
    compiler_params=_sc_params,
    scratch_types=[
        pltpu.VMEM((NCH_DEG, CH), jnp.int32),
        pltpu.VMEM((CH,), jnp.float32),
        pltpu.VMEM_SHARED((NP,), jnp.float32),
    ],
)
def _sc_degrees(idx2_hbm, zeros_hbm, out_hbm, idx_v, ones_v, acc_sh):
    c = lax.axis_index("c")
    s = lax.axis_index("s")
    base = s * RPT
    pltpu.sync_copy(idx2_hbm.at[c, s], idx_v)
    for i in range(CH // 16):
        ones_v[pl.ds(i * 16, 16)] = jnp.ones((16,), jnp.float32)
    pltpu.sync_copy(zeros_hbm.at[pl.ds(base, RPT)], acc_sh.at[pl.ds(base, RPT)])
    plsc.subcore_barrier()

    def body(j, carry):
        pltpu.sync_copy(ones_v, acc_sh.at[idx_v.at[j]], add=True)
        return carry

    lax.fori_loop(0, NCH_DEG, body, 0)
    plsc.subcore_barrier()
    pltpu.sync_copy(acc_sh.at[pl.ds(base, RPT)], out_hbm.at[c, pl.ds(base, RPT)])


@functools.partial(
    pl.kernel,
    out_type=jax.ShapeDtypeStruct((2, NP, G), jnp.float32),
    mesh=_mesh,
    compiler_params=_sc_params,
    scratch_types=[
        pltpu.VMEM((NCH_MAIN, CH), jnp.int32),
        pltpu.VMEM((NCH_MAIN, CH), jnp.int32),
        pltpu.VMEM((K, CH, G), jnp.float32),
        pltpu.VMEM_SHARED((NP, G), jnp.float32),
        pltpu.SemaphoreType.DMA((K,)),
    ],
)
def _sc_edge_agg(hp_hbm, src_hbm, dst_hbm, zeros_hbm, out_hbm,
                 sidx_v, didx_v, rows_v, acc_sh, sem_g):
    c = lax.axis_index("c")
    s = lax.axis_index("s")
    w = s * 2 + c
    base = s * RPT
    pltpu.sync_copy(src_hbm.at[w], sidx_v)
    pltpu.sync_copy(dst_hbm.at[w], didx_v)
    pltpu.sync_copy(zeros_hbm.at[pl.ds(base, RPT)], acc_sh.at[pl.ds(base, RPT)])
    plsc.subcore_barrier()

    for b in range(K):
        pltpu.async_copy(hp_hbm.at[sidx_v.at[b]], rows_v.at[b], sem_g.at[b])

    def grp(g, carry):
        for b in range(K):
            cch = g * K + b
            pltpu.make_async_copy(
                hp_hbm.at[sidx_v.at[cch]], rows_v.at[b], sem_g.at[b]).wait()
            pltpu.sync_copy(rows_v.at[b], acc_sh.at[didx_v.at[cch]], add=True)
            nch = cch + K

            @pl.when(nch < NCH_MAIN)
            def _():
                pltpu.async_copy(
                    hp_hbm.at[sidx_v.at[nch]], rows_v.at[b], sem_g.at[b])
        return carry

    lax.fori_loop(0, NGRP, grp, 0)
    plsc.subcore_barrier()
    pltpu.sync_copy(acc_sh.at[pl.ds(base, RPT)], out_hbm.at[c, pl.ds(base, RPT)])


def _tc_pre_body(xu_ref, xs_ref, w0a_ref, w0b_ref, nsr_ref, hp0_ref):
    h = jnp.dot(xu_ref[...], w0a_ref[...], preferred_element_type=jnp.float32)
    h += jnp.dot(xs_ref[...], w0b_ref[...], preferred_element_type=jnp.float32)
    hp0_ref[...] = h * nsr_ref[...]


_tc_pre = pl.pallas_call(
    _tc_pre_body,
    out_shape=jax.ShapeDtypeStruct((NP, G), jnp.float32),
)


def _tc_mid_body(agg_ref, ndr_ref, nsr_ref, b0_ref, hp1_ref):
    a = (agg_ref[0] + agg_ref[1]) * ndr_ref[...]
    h1 = jnp.maximum(a + b0_ref[...][None, :], 0.0)
    hp1_ref[...] = h1 * nsr_ref[...]


_tc_mid = pl.pallas_call(
    _tc_mid_body,
    out_shape=jax.ShapeDtypeStruct((NP, G), jnp.float32),
)


def _tc_post_body(agg_ref, ndr_ref, w1a_ref, w1b_ref, b1a_ref, b1b_ref,
                  xu_ref, xs_ref, out_ref):
    a = (agg_ref[0] + agg_ref[1]) * ndr_ref[...]
    beta = jnp.dot(a, w1a_ref[...], preferred_element_type=jnp.float32)
    beta += b1a_ref[...][None, :]
    gamma = jnp.dot(a, w1b_ref[...], preferred_element_type=jnp.float32)
    gamma += b1b_ref[...][None, :]
    out_ref[...] = beta * xu_ref[...] + gamma * xs_ref[...]


_tc_post = pl.pallas_call(
    _tc_post_body,
    out_shape=jax.ShapeDtypeStruct((NP, G), jnp.float32),
)


def kernel(x_u, x_s, edge_index, W0, b0, W1, b1):
    src = edge_index[0].astype(jnp.int32)
    dst = edge_index[1].astype(jnp.int32)
    pad = jnp.full((EP - E,), N, jnp.int32)
    srcp = jnp.concatenate([src, pad])
    dstp = jnp.concatenate([dst, pad])
    src_mb = srcp.reshape(32, NCH_MAIN, CH)
    dst_mb = dstp.reshape(32, NCH_MAIN, CH)
    idx2 = jnp.stack([dstp, srcp]).reshape(2, 16, NCH_DEG, CH)
    zeros_d = jnp.zeros((NP,), jnp.float32)
    zeros_f = jnp.zeros((NP, G), jnp.float32)
    xup = jnp.pad(x_u, ((0, NP - N), (0, 0)))
    xsp = jnp.pad(x_s, ((0, NP - N), (0, 0)))

    deg2 = _sc_degrees(idx2, zeros_d)
    ndr = jnp.broadcast_to(
        lax.rsqrt(jnp.maximum(deg2[0], 1.0))[:, None], (NP, G))
    nsr = jnp.broadcast_to(
        lax.rsqrt(jnp.maximum(deg2[1], 1.0))[:, None], (NP, G))

    hp0 = _tc_pre(xup, xsp, W0[:G], W0[G:], nsr)
    agg0 = _sc_edge_agg(hp0, src_mb, dst_mb, zeros_f)
    hp1 = _tc_mid(agg0, ndr, nsr, b0)
    agg1 = _sc_edge_agg(hp1, src_mb, dst_mb, zeros_f)
    pred = _tc_post(agg1, ndr, W1[:, :G], W1[:, G:], b1[:G], b1[G:], xup, xsp)
    return pred[:N]


# speedup vs baseline: 2.5397x; 2.5397x over previous
import functools

import jax
import jax.numpy as jnp
from jax import lax
from jax.experimental import pallas as pl
from jax.experimental.pallas import tpu as pltpu
from jax.experimental.pallas import tpu_sc as plsc

N = 10000
NP = 10240
G = 64
E = 320000
EP = 327680
CH = 128
NCH_MAIN = 80
NCH_DEG = 160
K = 4
NGRP = NCH_MAIN // K
RPT = NP // 16

_mesh = plsc.VectorSubcoreMesh(core_axis_name="c", subcore_axis_name="s")
_sc_params = pltpu.CompilerParams(use_tc_tiling_on_sc=False)


@functools.partial(
    pl.kernel,
    out_type=jax.ShapeDtypeStruct((2, NP), jnp.float32),
    mesh=_mesh,
    compiler_params=_sc_params,
    scratch_types=[
        pltpu.VMEM((NCH_DEG, CH), jnp.int32),
        pltpu.VMEM((CH,), jnp.float32),
        pltpu.VMEM_SHARED((NP,), jnp.float32),
    ],
)
def _sc_degrees(idx2_hbm, zeros_hbm, out_hbm, idx_v, ones_v, acc_sh):
    c = lax.axis_index("c")
    s = lax.axis_index("s")
    base = s * RPT
    pltpu.sync_copy(idx2_hbm.at[c, s], idx_v)
    for i in range(CH // 16):
        ones_v[pl.ds(i * 16, 16)] = jnp.ones((16,), jnp.float32)
    pltpu.sync_copy(zeros_hbm.at[pl.ds(base, RPT)], acc_sh.at[pl.ds(base, RPT)])
    plsc.subcore_barrier()

    def body(j, carry):
        pltpu.sync_copy(ones_v, acc_sh.at[idx_v.at[j]], add=True)
        return carry

    lax.fori_loop(0, NCH_DEG, body, 0)
    plsc.subcore_barrier()
    pltpu.sync_copy(acc_sh.at[pl.ds(base, RPT)], out_hbm.at[c, pl.ds(base, RPT)])


@functools.partial(
    pl.kernel,
    out_type=jax.ShapeDtypeStruct((2, NP, G), jnp.float32),
    mesh=_mesh,
    compiler_params=_sc_params,
    scratch_types=[
        pltpu.VMEM((NCH_MAIN, CH), jnp.int32),
        pltpu.VMEM((NCH_MAIN, CH), jnp.int32),
        pltpu.VMEM((K, CH, G), jnp.float32),
        pltpu.VMEM_SHARED((NP, G), jnp.float32),
        pltpu.SemaphoreType.DMA((K,)),
    ],
)
def _sc_edge_agg(hp_hbm, src_hbm, dst_hbm, zeros_hbm, out_hbm,
                 sidx_v, didx_v, rows_v, acc_sh, sem_g):
    c = lax.axis_index("c")
    s = lax.axis_index("s")
    w = s * 2 + c
    base = s * RPT
    pltpu.sync_copy(src_hbm.at[w], sidx_v)
    pltpu.sync_copy(dst_hbm.at[w], didx_v)
    pltpu.sync_copy(zeros_hbm.at[pl.ds(base, RPT)], acc_sh.at[pl.ds(base, RPT)])
    plsc.subcore_barrier()

    for b in range(K):
        pltpu.async_copy(hp_hbm.at[sidx_v.at[b]], rows_v.at[b], sem_g.at[b])

    def grp(g, carry):
        for b in range(K):
            cch = g * K + b
            pltpu.make_async_copy(
                hp_hbm.at[sidx_v.at[cch]], rows_v.at[b], sem_g.at[b]).wait()
            pltpu.sync_copy(rows_v.at[b], acc_sh.at[didx_v.at[cch]], add=True)
            nch = cch + K

            @pl.when(nch < NCH_MAIN)
            def _():
                pltpu.async_copy(
                    hp_hbm.at[sidx_v.at[nch]], rows_v.at[b], sem_g.at[b])
        return carry

    lax.fori_loop(0, NGRP, grp, 0)
    plsc.subcore_barrier()
    pltpu.sync_copy(acc_sh.at[pl.ds(base, RPT)], out_hbm.at[c, pl.ds(base, RPT)])


def _tc_pre_body(xu_ref, xs_ref, w0a_ref, w0b_ref, nsr_ref, hp0_ref):
    h = jnp.dot(xu_ref[...], w0a_ref[...], preferred_element_type=jnp.float32)
    h += jnp.dot(xs_ref[...], w0b_ref[...], preferred_element_type=jnp.float32)
    hp0_ref[...] = h * nsr_ref[...]


_tc_pre = pl.pallas_call(
    _tc_pre_body,
    out_shape=jax.ShapeDtypeStruct((NP, G), jnp.float32),
)


def _tc_mid_body(agg_ref, ndr_ref, nsr_ref, b0_ref, hp1_ref):
    a = (agg_ref[0] + agg_ref[1]) * ndr_ref[...]
    h1 = jnp.maximum(a + b0_ref[...][None, :], 0.0)
    hp1_ref[...] = h1 * nsr_ref[...]


_tc_mid = pl.pallas_call(
    _tc_mid_body,
    out_shape=jax.ShapeDtypeStruct((NP, G), jnp.float32),
)


def _tc_post_body(agg_ref, ndr_ref, w1a_ref, w1b_ref, b1a_ref, b1b_ref,
                  xu_ref, xs_ref, out_ref):
    a = (agg_ref[0] + agg_ref[1]) * ndr_ref[...]
    beta = jnp.dot(a, w1a_ref[...], preferred_element_type=jnp.float32)
    beta += b1a_ref[...][None, :]
    gamma = jnp.dot(a, w1b_ref[...], preferred_element_type=jnp.float32)
    gamma += b1b_ref[...][None, :]
    out_ref[...] = beta * xu_ref[...] + gamma * xs_ref[...]


_tc_post = pl.pallas_call(
    _tc_post_body,
    out_shape=jax.ShapeDtypeStruct((NP, G), jnp.float32),
)


def kernel(x_u, x_s, edge_index, W0, b0, W1, b1):
    src = edge_index[0].astype(jnp.int32)
    dst = edge_index[1].astype(jnp.int32)
    pad = N + (jnp.arange(EP - E, dtype=jnp.int32) % (NP - N))
    srcp = jnp.concatenate([src, pad])
    dstp = jnp.concatenate([dst, pad])
    src_mb = srcp.reshape(32, NCH_MAIN, CH)
    dst_mb = dstp.reshape(32, NCH_MAIN, CH)
    idx2 = jnp.stack([dstp, srcp]).reshape(2, 16, NCH_DEG, CH)
    zeros_d = jnp.zeros((NP,), jnp.float32)
    zeros_f = jnp.zeros((NP, G), jnp.float32)
    xup = jnp.pad(x_u, ((0, NP - N), (0, 0)))
    xsp = jnp.pad(x_s, ((0, NP - N), (0, 0)))

    deg2 = _sc_degrees(idx2, zeros_d)
    ndr = jnp.broadcast_to(
        lax.rsqrt(jnp.maximum(deg2[0], 1.0))[:, None], (NP, G))
    nsr = jnp.broadcast_to(
        lax.rsqrt(jnp.maximum(deg2[1], 1.0))[:, None], (NP, G))

    hp0 = _tc_pre(xup, xsp, W0[:G], W0[G:], nsr)
    agg0 = _sc_edge_agg(hp0, src_mb, dst_mb, zeros_f)
    hp1 = _tc_mid(agg0, ndr, nsr, b0)
    agg1 = _sc_edge_agg(hp1, src_mb, dst_mb, zeros_f)
    pred = _tc_post(agg1, ndr, W1[:, :G], W1[:, G:], b1[:G], b1[G:], xup, xsp)
    return pred[:N]


# speedup vs baseline: 2.7033x; 1.0644x over previous
import functools

import jax
import jax.numpy as jnp
from jax import lax
from jax.experimental import pallas as pl
from jax.experimental.pallas import tpu as pltpu
from jax.experimental.pallas import tpu_sc as plsc

N = 10000
NP = 10240
G = 64
E = 320000
EP = 327680
CH = 128
NCH_MAIN = 80
NCH_DEG = 160
K = 4
NGRP = NCH_MAIN // K
KD = 16
RPT = NP // 16

_mesh = plsc.VectorSubcoreMesh(core_axis_name="c", subcore_axis_name="s")
_sc_params = pltpu.CompilerParams(use_tc_tiling_on_sc=False)


@functools.partial(
    pl.kernel,
    out_type=jax.ShapeDtypeStruct((2, NP), jnp.float32),
    mesh=_mesh,
    compiler_params=_sc_params,
    scratch_types=[
        pltpu.VMEM((NCH_DEG, CH), jnp.int32),
        pltpu.VMEM((CH,), jnp.float32),
        pltpu.VMEM_SHARED((NP,), jnp.float32),
        pltpu.SemaphoreType.DMA,
    ],
)
def _sc_degrees(src_hbm, dst_hbm, zeros_hbm, out_hbm, idx_v, ones_v, acc_sh,
                sem):
    c = lax.axis_index("c")
    s = lax.axis_index("s")
    base = s * RPT

    @pl.when(c == 0)
    def _():
        pltpu.sync_copy(dst_hbm.at[2 * s], idx_v.at[pl.ds(0, NCH_MAIN)])
        pltpu.sync_copy(dst_hbm.at[2 * s + 1], idx_v.at[pl.ds(NCH_MAIN, NCH_MAIN)])

    @pl.when(c == 1)
    def _():
        pltpu.sync_copy(src_hbm.at[2 * s], idx_v.at[pl.ds(0, NCH_MAIN)])
        pltpu.sync_copy(src_hbm.at[2 * s + 1], idx_v.at[pl.ds(NCH_MAIN, NCH_MAIN)])

    for i in range(CH // 16):
        ones_v[pl.ds(i * 16, 16)] = jnp.ones((16,), jnp.float32)
    pltpu.sync_copy(zeros_hbm.at[pl.ds(base, RPT)], acc_sh.at[pl.ds(base, RPT)])
    plsc.subcore_barrier()

    def grp(g, carry):
        for b in range(KD):
            pltpu.async_copy(ones_v, acc_sh.at[idx_v.at[g * KD + b]], sem,
                             add=True)
        for b in range(KD):
            pltpu.make_async_copy(
                ones_v, acc_sh.at[idx_v.at[g * KD + b]], sem).wait()
        return carry

    lax.fori_loop(0, NCH_DEG // KD, grp, 0)
    plsc.subcore_barrier()
    pltpu.sync_copy(acc_sh.at[pl.ds(base, RPT)], out_hbm.at[c, pl.ds(base, RPT)])


@functools.partial(
    pl.kernel,
    out_type=jax.ShapeDtypeStruct((2, NP, G), jnp.float32),
    mesh=_mesh,
    compiler_params=_sc_params,
    scratch_types=[
        pltpu.VMEM((NCH_MAIN, CH), jnp.int32),
        pltpu.VMEM((NCH_MAIN, CH), jnp.int32),
        pltpu.VMEM((K, CH, G), jnp.float32),
        pltpu.VMEM_SHARED((NP, G), jnp.float32),
        pltpu.SemaphoreType.DMA((K,)),
    ],
)
def _sc_edge_agg(hp_hbm, src_hbm, dst_hbm, zeros_hbm, out_hbm,
                 sidx_v, didx_v, rows_v, acc_sh, sem_g):
    c = lax.axis_index("c")
    s = lax.axis_index("s")
    w = s * 2 + c
    base = s * RPT
    pltpu.sync_copy(src_hbm.at[w], sidx_v)
    pltpu.sync_copy(dst_hbm.at[w], didx_v)
    pltpu.sync_copy(zeros_hbm.at[pl.ds(base, RPT)], acc_sh.at[pl.ds(base, RPT)])
    plsc.subcore_barrier()

    for b in range(K):
        pltpu.async_copy(hp_hbm.at[sidx_v.at[b]], rows_v.at[b], sem_g.at[b])

    def grp(g, carry):
        for b in range(K):
            cch = g * K + b
            pltpu.make_async_copy(
                hp_hbm.at[sidx_v.at[cch]], rows_v.at[b], sem_g.at[b]).wait()
            pltpu.sync_copy(rows_v.at[b], acc_sh.at[didx_v.at[cch]], add=True)
            nch = cch + K

            @pl.when(nch < NCH_MAIN)
            def _():
                pltpu.async_copy(
                    hp_hbm.at[sidx_v.at[nch]], rows_v.at[b], sem_g.at[b])
        return carry

    lax.fori_loop(0, NGRP, grp, 0)
    plsc.subcore_barrier()
    pltpu.sync_copy(acc_sh.at[pl.ds(base, RPT)], out_hbm.at[c, pl.ds(base, RPT)])


def _tc_pre_body(xu_ref, xs_ref, w0a_ref, w0b_ref, dout_ref, hp0_ref):
    h = jnp.dot(xu_ref[...], w0a_ref[...], preferred_element_type=jnp.float32)
    h += jnp.dot(xs_ref[...], w0b_ref[...], preferred_element_type=jnp.float32)
    ns = lax.rsqrt(jnp.maximum(dout_ref[...], 1.0))
    hp0_ref[0:N, :] = h * ns[0:N]
    hp0_ref[N:NP, :] = jnp.zeros((NP - N, G), jnp.float32)


_tc_pre = pl.pallas_call(
    _tc_pre_body,
    out_shape=jax.ShapeDtypeStruct((NP, G), jnp.float32),
)


def _tc_mid_body(agg_ref, din_ref, dout_ref, b0_ref, hp1_ref):
    nd = lax.rsqrt(jnp.maximum(din_ref[...], 1.0))
    ns = lax.rsqrt(jnp.maximum(dout_ref[...], 1.0))
    a = (agg_ref[0] + agg_ref[1]) * nd
    h1 = jnp.maximum(a + b0_ref[...][None, :], 0.0)
    hp1_ref[...] = h1 * ns


_tc_mid = pl.pallas_call(
    _tc_mid_body,
    out_shape=jax.ShapeDtypeStruct((NP, G), jnp.float32),
)


def _tc_post_body(agg_ref, din_ref, w1a_ref, w1b_ref, b1a_ref, b1b_ref,
                  xu_ref, xs_ref, out_ref):
    nd = lax.rsqrt(jnp.maximum(din_ref[...], 1.0))
    a = ((agg_ref[0] + agg_ref[1]) * nd)[0:N, :]
    beta = jnp.dot(a, w1a_ref[...], preferred_element_type=jnp.float32)
    beta += b1a_ref[...][None, :]
    gamma = jnp.dot(a, w1b_ref[...], preferred_element_type=jnp.float32)
    gamma += b1b_ref[...][None, :]
    out_ref[...] = beta * xu_ref[...] + gamma * xs_ref[...]


_tc_post = pl.pallas_call(
    _tc_post_body,
    out_shape=jax.ShapeDtypeStruct((N, G), jnp.float32),
)


def kernel(x_u, x_s, edge_index, W0, b0, W1, b1):
    src = edge_index[0].astype(jnp.int32)
    dst = edge_index[1].astype(jnp.int32)
    pad = N + (jnp.arange(EP - E, dtype=jnp.int32) % (NP - N))
    srcp = jnp.concatenate([src, pad])
    dstp = jnp.concatenate([dst, pad])
    src_mb = srcp.reshape(32, NCH_MAIN, CH)
    dst_mb = dstp.reshape(32, NCH_MAIN, CH)
    zeros_d = jnp.zeros((NP,), jnp.float32)
    zeros_f = jnp.zeros((NP, G), jnp.float32)

    deg2 = _sc_degrees(src_mb, dst_mb, zeros_d)
    din = deg2[0][:, None]
    dout = deg2[1][:, None]

    hp0 = _tc_pre(x_u, x_s, W0[:G], W0[G:], dout)
    agg0 = _sc_edge_agg(hp0, src_mb, dst_mb, zeros_f)
    hp1 = _tc_mid(agg0, din, dout, b0)
    agg1 = _sc_edge_agg(hp1, src_mb, dst_mb, zeros_f)
    return _tc_post(agg1, din, W1[:, :G], W1[:, G:], b1[:G], b1[G:], x_u, x_s)
